# Initial kernel scaffold; baseline (speedup 1.0000x reference)
import jax
import jax.numpy as jnp
from jax.experimental import pallas as pl


def kernel(x, table_2, table_3, table_4):
    raise NotImplementedError("write your pallas kernel here")


# speedup vs baseline: 1.4549x; 1.4549x over previous
import functools

import jax
import jax.numpy as jnp
from jax import lax
from jax.experimental import pallas as pl
from jax.experimental.pallas import tpu as pltpu
from jax.experimental.pallas import tpu_sc as plsc

B, S = 4096, 200
TABLE_SIZE = 1000000
MULT = 2654435761

NC, NS = 2, 16
NW = NC * NS
TOTAL = B * S
PER_W = TOTAL // NW
CHUNK = 6400
NCHUNK = PER_W // CHUNK


def _hash_body(x_ref, idx_ref):
    x = x_ref[...].astype(jnp.uint32)
    mult = jnp.uint32(MULT)
    ts = jnp.uint32(TABLE_SIZE)
    z = jnp.zeros((B, 1), jnp.uint32)
    xs = jnp.concatenate([z, x[:, :-1]], axis=1)
    h2 = xs * mult + x
    h2s = jnp.concatenate([z, h2[:, :-1]], axis=1)
    h3 = h2s * mult + x
    h3s = jnp.concatenate([z, h3[:, :-1]], axis=1)
    h4 = h3s * mult + x
    idx_ref[0] = (h2 % ts).astype(jnp.int32)
    idx_ref[1] = (h3 % ts).astype(jnp.int32)
    idx_ref[2] = (h4 % ts).astype(jnp.int32)


_hash = pl.pallas_call(
    _hash_body,
    out_shape=jax.ShapeDtypeStruct((3, B, S), jnp.int32),
)


def _gather_body(idx_hbm, t2, t3, t4, out_hbm, idx_v, val_v, sem):
    wid = lax.axis_index("s") * NC + lax.axis_index("c")
    base = wid * PER_W
    tabs = [t2, t3, t4]
    for n in range(3):
        for k in range(NCHUNK):
            off = n * TOTAL + base + k * CHUNK
            pltpu.sync_copy(idx_hbm.at[pl.ds(off, CHUNK)], idx_v)
            pltpu.async_copy(tabs[n].at[idx_v], val_v, sem).wait()
            pltpu.sync_copy(val_v, out_hbm.at[pl.ds(off, CHUNK)])


@functools.cache
def _gather():
    return pl.kernel(
        _gather_body,
        out_type=jax.ShapeDtypeStruct((3 * TOTAL,), jnp.float32),
        mesh=plsc.VectorSubcoreMesh(core_axis_name="c", subcore_axis_name="s",
                                    num_cores=NC, num_subcores=NS),
        scratch_types=[
            pltpu.VMEM((CHUNK,), jnp.int32),
            pltpu.VMEM((CHUNK,), jnp.float32),
            pltpu.SemaphoreType.DMA,
        ],
    )


@jax.jit
def kernel(x, table_2, table_3, table_4):
    idx = _hash(x).reshape(3 * TOTAL)
    out = _gather()(idx, table_2, table_3, table_4)
    return out.reshape(3, B, S)


# speedup vs baseline: 1.5461x; 1.0627x over previous
import functools

import jax
import jax.numpy as jnp
from jax import lax
from jax.experimental import pallas as pl
from jax.experimental.pallas import tpu as pltpu
from jax.experimental.pallas import tpu_sc as plsc

B, S = 4096, 200
TABLE_SIZE = 1000000
MULT = 2654435761

NC, NS = 2, 16
NW = NC * NS
TOTAL = B * S
PER_W = TOTAL // NW
CHUNK = 6400
NCHUNK = PER_W // CHUNK


def _hash_body(x_ref, idx_ref):
    x = x_ref[...].astype(jnp.uint32)
    mult = jnp.uint32(MULT)
    ts = jnp.uint32(TABLE_SIZE)
    z = jnp.zeros((B, 1), jnp.uint32)
    xs = jnp.concatenate([z, x[:, :-1]], axis=1)
    h2 = xs * mult + x
    h2s = jnp.concatenate([z, h2[:, :-1]], axis=1)
    h3 = h2s * mult + x
    h3s = jnp.concatenate([z, h3[:, :-1]], axis=1)
    h4 = h3s * mult + x
    idx_ref[0] = (h2 % ts).astype(jnp.int32)
    idx_ref[1] = (h3 % ts).astype(jnp.int32)
    idx_ref[2] = (h4 % ts).astype(jnp.int32)


_hash = pl.pallas_call(
    _hash_body,
    out_shape=jax.ShapeDtypeStruct((3, B, S), jnp.int32),
)


def _gather_body(idx_hbm, t2, t3, t4, out_hbm,
                 idx_v0, idx_v1, val_v0, val_v1,
                 si0, si1, sg0, sg1, ss0, ss1):
    wid = lax.axis_index("s") * NC + lax.axis_index("c")
    base = wid * PER_W
    tabs = [t2, t3, t4]
    idx_b = [idx_v0, idx_v1]
    val_b = [val_v0, val_v1]
    si = [si0, si1]
    sg = [sg0, sg1]
    ss = [ss0, ss1]
    chunks = [(n, k) for n in range(3) for k in range(NCHUNK)]
    n_chunks = len(chunks)

    def off(i):
        n, k = chunks[i]
        return n * TOTAL + base + k * CHUNK

    h_idx = [None] * n_chunks
    h_g = [None] * n_chunks
    h_s = [None] * n_chunks
    h_idx[0] = pltpu.async_copy(idx_hbm.at[pl.ds(off(0), CHUNK)], idx_b[0], si[0])
    for i in range(n_chunks):
        b = i % 2
        if i + 1 < n_chunks:
            h_idx[i + 1] = pltpu.async_copy(
                idx_hbm.at[pl.ds(off(i + 1), CHUNK)], idx_b[1 - b], si[1 - b])
        h_idx[i].wait()
        h_g[i] = pltpu.async_copy(tabs[chunks[i][0]].at[idx_b[b]], val_b[b], sg[b])
        h_g[i].wait()
        h_s[i] = pltpu.async_copy(val_b[b], out_hbm.at[pl.ds(off(i), CHUNK)], ss[b])
        if i >= 1:
            h_s[i - 1].wait()
    h_s[n_chunks - 1].wait()


@functools.cache
def _gather():
    return pl.kernel(
        _gather_body,
        out_type=jax.ShapeDtypeStruct((3 * TOTAL,), jnp.float32),
        mesh=plsc.VectorSubcoreMesh(core_axis_name="c", subcore_axis_name="s",
                                    num_cores=NC, num_subcores=NS),
        scratch_types=[
            pltpu.VMEM((CHUNK,), jnp.int32),
            pltpu.VMEM((CHUNK,), jnp.int32),
            pltpu.VMEM((CHUNK,), jnp.float32),
            pltpu.VMEM((CHUNK,), jnp.float32),
            pltpu.SemaphoreType.DMA,
            pltpu.SemaphoreType.DMA,
            pltpu.SemaphoreType.DMA,
            pltpu.SemaphoreType.DMA,
            pltpu.SemaphoreType.DMA,
            pltpu.SemaphoreType.DMA,
        ],
    )


@jax.jit
def kernel(x, table_2, table_3, table_4):
    idx = _hash(x).reshape(3 * TOTAL)
    out = _gather()(idx, table_2, table_3, table_4)
    return out.reshape(3, B, S)


# speedup vs baseline: 1.7849x; 1.1545x over previous
import functools

import jax
import jax.numpy as jnp
from jax import lax
from jax.experimental import pallas as pl
from jax.experimental.pallas import tpu as pltpu
from jax.experimental.pallas import tpu_sc as plsc

B, S = 4096, 200
TABLE_SIZE = 1000000
MULT = 2654435761

NC, NS = 2, 16
NW = NC * NS
TOTAL = B * S
ROWS_W = B // NW
RB = 32
NB = ROWS_W // RB
BLK = RB * S
PAD = 8
NCH = 13


MULT2 = (MULT * MULT) % (1 << 32)
MULT3 = (MULT * MULT * MULT) % (1 << 32)


def _row_hash_body(xb, i2, i3, i4):
    m1 = jnp.uint32(MULT)
    m2 = jnp.uint32(MULT2)
    m3 = jnp.uint32(MULT3)
    ts = jnp.uint32(TABLE_SIZE)
    lanes = lax.iota(jnp.uint32, 16)
    ge1 = lanes >= jnp.uint32(1)
    ge2 = lanes >= jnp.uint32(2)
    ge3 = lanes >= jnp.uint32(3)

    def body(r, carry):
        p0 = PAD + r * S
        q0 = r * S
        for c in range(NCH):
            s0 = 184 if c == NCH - 1 else 16 * c
            p = p0 + s0
            q = q0 + s0
            t = xb[pl.ds(p, 16)].astype(jnp.uint32)
            tm1 = xb[pl.ds(p - 1, 16)].astype(jnp.uint32)
            tm2 = xb[pl.ds(p - 2, 16)].astype(jnp.uint32)
            tm3 = xb[pl.ds(p - 3, 16)].astype(jnp.uint32)
            if c == 0:
                tm1 = jnp.where(ge1, tm1, jnp.uint32(0))
                tm2 = jnp.where(ge2, tm2, jnp.uint32(0))
                tm3 = jnp.where(ge3, tm3, jnp.uint32(0))
            h2 = tm1 * m1 + t
            h3 = tm2 * m2 + h2
            h4 = tm3 * m3 + h3
            i2[pl.ds(q, 16)] = (h2 % ts).astype(jnp.int32)
            i3[pl.ds(q, 16)] = (h3 % ts).astype(jnp.int32)
            i4[pl.ds(q, 16)] = (h4 % ts).astype(jnp.int32)
        return carry

    return body


def _fused_body(x_hbm, t2, t3, t4, out_hbm,
                xb0, xb1,
                i20, i30, i40, i21, i31, i41,
                v20, v30, v40, v21, v31, v41,
                sx0, sx1, sg0, sg1, ss0, ss1):
    wid = lax.axis_index("s") * NC + lax.axis_index("c")
    row0 = wid * ROWS_W
    tabs = [t2, t3, t4]
    xb = [xb0, xb1]
    idx = [[i20, i30, i40], [i21, i31, i41]]
    val = [[v20, v30, v40], [v21, v31, v41]]
    sx = [sx0, sx1]
    sg = [sg0, sg1]
    ss = [ss0, ss1]

    def x_src(j):
        return x_hbm.at[pl.ds((row0 + j * RB) * S, BLK)]

    def out_dst(j, n):
        return out_hbm.at[pl.ds(n * TOTAL + (row0 + j * RB) * S, BLK)]

    hx = [None] * NB
    hg = [None] * NB
    hs = [None] * NB
    hx[0] = pltpu.async_copy(x_src(0), xb[0].at[pl.ds(PAD, BLK)], sx[0])
    for j in range(NB):
        b = j % 2
        hx[j].wait()
        if j + 1 < NB:
            hx[j + 1] = pltpu.async_copy(
                x_src(j + 1), xb[1 - b].at[pl.ds(PAD, BLK)], sx[1 - b])
        body = _row_hash_body(xb[b], *idx[b])
        lax.fori_loop(0, RB, body, 0)
        if j >= 1:
            for h in hg[j - 1]:
                h.wait()
            hs[j - 1] = [
                pltpu.async_copy(val[1 - b][n], out_dst(j - 1, n), ss[1 - b])
                for n in range(3)]
        if j >= 2:
            for h in hs[j - 2]:
                h.wait()
        hg[j] = [
            pltpu.async_copy(tabs[n].at[idx[b][n]], val[b][n], sg[b])
            for n in range(3)]
    bl = (NB - 1) % 2
    for h in hg[NB - 1]:
        h.wait()
    hs[NB - 1] = [
        pltpu.async_copy(val[bl][n], out_dst(NB - 1, n), ss[bl])
        for n in range(3)]
    for h in hs[NB - 2]:
        h.wait()
    for h in hs[NB - 1]:
        h.wait()


@functools.cache
def _fused():
    return pl.kernel(
        _fused_body,
        out_type=jax.ShapeDtypeStruct((3 * TOTAL,), jnp.float32),
        mesh=plsc.VectorSubcoreMesh(core_axis_name="c", subcore_axis_name="s",
                                    num_cores=NC, num_subcores=NS),
        scratch_types=(
            [pltpu.VMEM((PAD + BLK,), jnp.int32) for _ in range(2)]
            + [pltpu.VMEM((BLK,), jnp.int32) for _ in range(6)]
            + [pltpu.VMEM((BLK,), jnp.float32) for _ in range(6)]
            + [pltpu.SemaphoreType.DMA for _ in range(6)]
        ),
    )


@jax.jit
def kernel(x, table_2, table_3, table_4):
    out = _fused()(x.reshape(-1), table_2, table_3, table_4)
    return out.reshape(3, B, S)


# speedup vs baseline: 1.7925x; 1.0043x over previous
import functools

import jax
import jax.numpy as jnp
from jax import lax
from jax.experimental import pallas as pl
from jax.experimental.pallas import tpu as pltpu
from jax.experimental.pallas import tpu_sc as plsc

B, S = 4096, 200
TABLE_SIZE = 1000000
MULT = 2654435761

NC, NS = 2, 16
NW = NC * NS
TOTAL = B * S
ROWS_W = B // NW
RB = 32
NB = ROWS_W // RB
BLK = RB * S
PAD = 8
NCH = 13


MULT2 = (MULT * MULT) % (1 << 32)
MULT3 = (MULT * MULT * MULT) % (1 << 32)


def _row_hash_body(xb, i2, i3, i4):
    m1 = jnp.uint32(MULT)
    m2 = jnp.uint32(MULT2)
    m3 = jnp.uint32(MULT3)
    ts = jnp.uint32(TABLE_SIZE)
    lanes = lax.iota(jnp.uint32, 16)
    ge1 = lanes >= jnp.uint32(1)
    ge2 = lanes >= jnp.uint32(2)
    ge3 = lanes >= jnp.uint32(3)

    def body(r, carry):
        p0 = PAD + r * S
        q0 = r * S
        for c in range(NCH):
            s0 = 184 if c == NCH - 1 else 16 * c
            p = p0 + s0
            q = q0 + s0
            t = xb[pl.ds(p, 16)].astype(jnp.uint32)
            tm1 = xb[pl.ds(p - 1, 16)].astype(jnp.uint32)
            tm2 = xb[pl.ds(p - 2, 16)].astype(jnp.uint32)
            tm3 = xb[pl.ds(p - 3, 16)].astype(jnp.uint32)
            if c == 0:
                tm1 = jnp.where(ge1, tm1, jnp.uint32(0))
                tm2 = jnp.where(ge2, tm2, jnp.uint32(0))
                tm3 = jnp.where(ge3, tm3, jnp.uint32(0))
            h2 = tm1 * m1 + t
            h3 = tm2 * m2 + h2
            h4 = tm3 * m3 + h3
            i2[pl.ds(q, 16)] = (h2 % ts).astype(jnp.int32)
            i3[pl.ds(q, 16)] = (h3 % ts).astype(jnp.int32)
            i4[pl.ds(q, 16)] = (h4 % ts).astype(jnp.int32)
        return carry

    return body


def _fused_body(x_hbm, t2, t3, t4, out_hbm,
                xb0, xb1,
                i20, i30, i40, i21, i31, i41,
                v20, v30, v40, v21, v31, v41,
                sx0, sx1, sg0, sg1, ss0, ss1):
    wid = lax.axis_index("s") * NC + lax.axis_index("c")
    row0 = wid * ROWS_W
    tabs = [t2, t3, t4]
    xb = [xb0, xb1]
    idx = [[i20, i30, i40], [i21, i31, i41]]
    val = [[v20, v30, v40], [v21, v31, v41]]
    sx = [sx0, sx1]
    sg = [sg0, sg1]
    ss = [ss0, ss1]

    def x_src(j):
        return x_hbm.at[pl.ds((row0 + j * RB) * S, BLK)]

    def out_dst(j, n):
        return out_hbm.at[pl.ds(n * TOTAL + (row0 + j * RB) * S, BLK)]

    hx = [None] * NB
    hg = [None] * NB
    hs = [None] * NB
    hx[0] = pltpu.async_copy(x_src(0), xb[0].at[pl.ds(PAD, BLK)], sx[0])
    for j in range(NB):
        b = j % 2
        hx[j].wait()
        if j + 1 < NB:
            hx[j + 1] = pltpu.async_copy(
                x_src(j + 1), xb[1 - b].at[pl.ds(PAD, BLK)], sx[1 - b])
        body = _row_hash_body(xb[b], *idx[b])
        lax.fori_loop(0, RB, body, 0)
        if j >= 2:
            for h in hs[j - 2]:
                h.wait()
        hg[j] = [
            pltpu.async_copy(tabs[n].at[idx[b][n]], val[b][n], sg[b])
            for n in range(3)]
        if j >= 1:
            for h in hg[j - 1]:
                h.wait()
            hs[j - 1] = [
                pltpu.async_copy(val[1 - b][n], out_dst(j - 1, n), ss[1 - b])
                for n in range(3)]
    bl = (NB - 1) % 2
    for h in hg[NB - 1]:
        h.wait()
    hs[NB - 1] = [
        pltpu.async_copy(val[bl][n], out_dst(NB - 1, n), ss[bl])
        for n in range(3)]
    for h in hs[NB - 2]:
        h.wait()
    for h in hs[NB - 1]:
        h.wait()


@functools.cache
def _fused():
    return pl.kernel(
        _fused_body,
        out_type=jax.ShapeDtypeStruct((3 * TOTAL,), jnp.float32),
        mesh=plsc.VectorSubcoreMesh(core_axis_name="c", subcore_axis_name="s",
                                    num_cores=NC, num_subcores=NS),
        scratch_types=(
            [pltpu.VMEM((PAD + BLK,), jnp.int32) for _ in range(2)]
            + [pltpu.VMEM((BLK,), jnp.int32) for _ in range(6)]
            + [pltpu.VMEM((BLK,), jnp.float32) for _ in range(6)]
            + [pltpu.SemaphoreType.DMA for _ in range(6)]
        ),
    )


@jax.jit
def kernel(x, table_2, table_3, table_4):
    out = _fused()(x.reshape(-1), table_2, table_3, table_4)
    return out.reshape(3, B, S)


# speedup vs baseline: 1.8049x; 1.0069x over previous
import functools

import jax
import jax.numpy as jnp
from jax import lax
from jax.experimental import pallas as pl
from jax.experimental.pallas import tpu as pltpu
from jax.experimental.pallas import tpu_sc as plsc

B, S = 4096, 200
TABLE_SIZE = 1000000
MULT = 2654435761

NC, NS = 2, 16
NW = NC * NS
TOTAL = B * S
ROWS_W = B // NW
RB = 16
NB = ROWS_W // RB
BLK = RB * S
PAD = 8
NCH = 13


MULT2 = (MULT * MULT) % (1 << 32)
MULT3 = (MULT * MULT * MULT) % (1 << 32)


def _row_hash_body(xb, i2, i3, i4):
    m1 = jnp.uint32(MULT)
    m2 = jnp.uint32(MULT2)
    m3 = jnp.uint32(MULT3)
    ts = jnp.uint32(TABLE_SIZE)
    lanes = lax.iota(jnp.uint32, 16)
    ge1 = lanes >= jnp.uint32(1)
    ge2 = lanes >= jnp.uint32(2)
    ge3 = lanes >= jnp.uint32(3)

    def body(r, carry):
        p0 = PAD + r * S
        q0 = r * S
        for c in range(NCH):
            s0 = 184 if c == NCH - 1 else 16 * c
            p = p0 + s0
            q = q0 + s0
            t = xb[pl.ds(p, 16)].astype(jnp.uint32)
            tm1 = xb[pl.ds(p - 1, 16)].astype(jnp.uint32)
            tm2 = xb[pl.ds(p - 2, 16)].astype(jnp.uint32)
            tm3 = xb[pl.ds(p - 3, 16)].astype(jnp.uint32)
            if c == 0:
                tm1 = jnp.where(ge1, tm1, jnp.uint32(0))
                tm2 = jnp.where(ge2, tm2, jnp.uint32(0))
                tm3 = jnp.where(ge3, tm3, jnp.uint32(0))
            h2 = tm1 * m1 + t
            h3 = tm2 * m2 + h2
            h4 = tm3 * m3 + h3
            i2[pl.ds(q, 16)] = (h2 % ts).astype(jnp.int32)
            i3[pl.ds(q, 16)] = (h3 % ts).astype(jnp.int32)
            i4[pl.ds(q, 16)] = (h4 % ts).astype(jnp.int32)
        return carry

    return body


def _fused_body(x_hbm, t2, t3, t4, out_hbm,
                xb0, xb1,
                i20, i30, i40, i21, i31, i41,
                v20, v30, v40, v21, v31, v41,
                sx0, sx1, sg0, sg1, ss0, ss1):
    wid = lax.axis_index("s") * NC + lax.axis_index("c")
    row0 = wid * ROWS_W
    tabs = [t2, t3, t4]
    xb = [xb0, xb1]
    idx = [[i20, i30, i40], [i21, i31, i41]]
    val = [[v20, v30, v40], [v21, v31, v41]]
    sx = [sx0, sx1]
    sg = [sg0, sg1]
    ss = [ss0, ss1]

    def x_src(j):
        return x_hbm.at[pl.ds((row0 + j * RB) * S, BLK)]

    def out_dst(j, n):
        return out_hbm.at[pl.ds(n * TOTAL + (row0 + j * RB) * S, BLK)]

    hx = [None] * NB
    hg = [None] * NB
    hs = [None] * NB
    hx[0] = pltpu.async_copy(x_src(0), xb[0].at[pl.ds(PAD, BLK)], sx[0])
    for j in range(NB):
        b = j % 2
        hx[j].wait()
        if j + 1 < NB:
            hx[j + 1] = pltpu.async_copy(
                x_src(j + 1), xb[1 - b].at[pl.ds(PAD, BLK)], sx[1 - b])
        body = _row_hash_body(xb[b], *idx[b])
        lax.fori_loop(0, RB, body, 0)
        if j >= 2:
            for h in hs[j - 2]:
                h.wait()
        hg[j] = [
            pltpu.async_copy(tabs[n].at[idx[b][n]], val[b][n], sg[b])
            for n in range(3)]
        if j >= 1:
            for h in hg[j - 1]:
                h.wait()
            hs[j - 1] = [
                pltpu.async_copy(val[1 - b][n], out_dst(j - 1, n), ss[1 - b])
                for n in range(3)]
    bl = (NB - 1) % 2
    for h in hg[NB - 1]:
        h.wait()
    hs[NB - 1] = [
        pltpu.async_copy(val[bl][n], out_dst(NB - 1, n), ss[bl])
        for n in range(3)]
    for h in hs[NB - 2]:
        h.wait()
    for h in hs[NB - 1]:
        h.wait()


@functools.cache
def _fused():
    return pl.kernel(
        _fused_body,
        out_type=jax.ShapeDtypeStruct((3 * TOTAL,), jnp.float32),
        mesh=plsc.VectorSubcoreMesh(core_axis_name="c", subcore_axis_name="s",
                                    num_cores=NC, num_subcores=NS),
        scratch_types=(
            [pltpu.VMEM((PAD + BLK,), jnp.int32) for _ in range(2)]
            + [pltpu.VMEM((BLK,), jnp.int32) for _ in range(6)]
            + [pltpu.VMEM((BLK,), jnp.float32) for _ in range(6)]
            + [pltpu.SemaphoreType.DMA for _ in range(6)]
        ),
    )


@jax.jit
def kernel(x, table_2, table_3, table_4):
    out = _fused()(x.reshape(-1), table_2, table_3, table_4)
    return out.reshape(3, B, S)
